# baseline (speedup 1.0000x reference)
import functools
import jax
import jax.numpy as jnp
from jax import lax
from jax.experimental import pallas as pl
from jax.experimental.pallas import tpu as pltpu
from jax.experimental.pallas import tpu_sc as plsc

N_NODES = 10000
N_PAD = 10240
JUNK = N_PAD - 1
R_PAD = 208
T_DUMMY = 200
W = 144
NT = 16
BE = 32
NB = 408
CH = 8
ALPHA = 0.2


def _sc_logits(qa, qb, qc, idx):
    mesh = plsc.VectorSubcoreMesh(core_axis_name="c", subcore_axis_name="s")

    @functools.partial(
        pl.kernel,
        out_type=jax.ShapeDtypeStruct((2, NT, NB, BE), jnp.float32),
        mesh=mesh,
        compiler_params=pltpu.CompilerParams(
            use_tc_tiling_on_sc=False, needs_layout_passes=False),
        scratch_types=dict(
            qa_t=pltpu.VMEM((2 * N_PAD,), jnp.float32),
            qb_t=pltpu.VMEM((2 * N_PAD,), jnp.float32),
            qc_t=pltpu.VMEM((R_PAD,), jnp.float32),
            ib=pltpu.VMEM((CH, 5, BE), jnp.int32),
            eb=pltpu.VMEM((CH, BE), jnp.float32),
        ),
    )
    def k(qa_h, qb_h, qc_h, idx_h, e_h, qa_t, qb_t, qc_t, ib, eb):
        c = lax.axis_index("c")
        s = lax.axis_index("s")
        pltpu.sync_copy(qa_h.at[c], qa_t)
        pltpu.sync_copy(qb_h.at[c], qb_t)
        pltpu.sync_copy(qc_h.at[c], qc_t)

        def chunk(g, _):
            pltpu.sync_copy(idx_h.at[s, pl.ds(g * CH, CH)], ib)

            def batch(b, _):
                def grp(h, _):
                    sl = pl.ds(h * 16, 16)
                    sv = (plsc.load_gather(qa_t, [ib[b, 0, sl]])
                          + plsc.load_gather(qb_t, [ib[b, 1, sl]])
                          + plsc.load_gather(qc_t, [ib[b, 2, sl]])
                          + plsc.load_gather(qc_t, [ib[b, 3, sl]]))
                    eb[b, sl] = jnp.exp(-jnp.maximum(sv, ALPHA * sv))
                    return 0

                return lax.fori_loop(0, BE // 16, grp, 0)

            lax.fori_loop(0, CH, batch, 0)
            pltpu.sync_copy(eb, e_h.at[c, s, pl.ds(g * CH, CH)])
            return 0

        lax.fori_loop(0, NB // CH, chunk, 0)

    return k(qa, qb, qc, idx)


def _sc_scatter(bt, ct, ev, idx):
    mesh = plsc.VectorSubcoreMesh(core_axis_name="c", subcore_axis_name="s")
    rows_per = N_PAD // NT

    @functools.partial(
        pl.kernel,
        out_type=jax.ShapeDtypeStruct((2, N_PAD, W), jnp.float32),
        mesh=mesh,
        compiler_params=pltpu.CompilerParams(
            use_tc_tiling_on_sc=False, needs_layout_passes=False),
        scratch_types=dict(
            ib=pltpu.VMEM((CH, 5, BE), jnp.int32),
            eb=pltpu.VMEM((CH, BE), jnp.float32),
            br=pltpu.VMEM((2, BE, W), jnp.float32),
            c0=pltpu.VMEM((2, BE, W), jnp.float32),
            c1=pltpu.VMEM((2, BE, W), jnp.float32),
            zb=pltpu.VMEM((BE, W), jnp.float32),
            acc=pltpu.VMEM_SHARED((N_PAD, W), jnp.float32),
            gsem=pltpu.SemaphoreType.DMA((2,)),
        ),
    )
    def k(bt_h, ct_h, ev_h, idx_h, out_h, ib, eb, br, c0, c1, zb, acc, gsem):
        c = lax.axis_index("c")
        s = lax.axis_index("s")

        def zi(i, _):
            for j in range(W // 16):
                zb[i, pl.ds(j * 16, 16)] = jnp.zeros((16,), jnp.float32)
            return 0

        lax.fori_loop(0, BE, zi, 0)
        for r in range(rows_per // BE):
            pltpu.sync_copy(zb, acc.at[pl.ds(s * rows_per + r * BE, BE)])
        plsc.subcore_barrier()

        def fire(b, p):
            pltpu.async_copy(bt_h.at[c].at[ib.at[b, 1]], br.at[p], gsem.at[p])
            pltpu.async_copy(ct_h.at[c].at[ib.at[b, 2]], c0.at[p], gsem.at[p])
            pltpu.async_copy(ct_h.at[c].at[ib.at[b, 3]], c1.at[p], gsem.at[p])

        def drain(b, p):
            pltpu.make_async_copy(bt_h.at[c].at[ib.at[b, 1]], br.at[p],
                                  gsem.at[p]).wait()
            pltpu.make_async_copy(ct_h.at[c].at[ib.at[b, 2]], c0.at[p],
                                  gsem.at[p]).wait()
            pltpu.make_async_copy(ct_h.at[c].at[ib.at[b, 3]], c1.at[p],
                                  gsem.at[p]).wait()

        def chunk(g, _):
            pltpu.sync_copy(idx_h.at[s, pl.ds(g * CH, CH)], ib)
            pltpu.sync_copy(ev_h.at[c, s, pl.ds(g * CH, CH)], eb)
            fire(0, 0)

            def batch(b, _):
                p = lax.rem(b, 2)

                @pl.when(b + 1 < CH)
                def _():
                    fire(b + 1, 1 - p)

                drain(b, p)

                def grp(h, _):
                    e16 = eb[b, pl.ds(h * 16, 16)]
                    for j in range(16):
                        i = h * 16 + j
                        e_i = e16[j]
                        for kk in range(W // 16):
                            sl = pl.ds(kk * 16, 16)
                            br[p, i, sl] = (br[p, i, sl] + c0[p, i, sl]
                                            + c1[p, i, sl]) * e_i
                    return 0

                lax.fori_loop(0, BE // 16, grp, 0)
                pltpu.sync_copy(br.at[p], acc.at[ib.at[b, 4]], add=True)
                return 0

            lax.fori_loop(0, CH, batch, 0)
            return 0

        lax.fori_loop(0, NB // CH, chunk, 0)
        plsc.subcore_barrier()
        pltpu.sync_copy(acc.at[pl.ds(s * rows_per, rows_per)],
                        out_h.at[c].at[pl.ds(s * rows_per, rows_per)])

    return k(bt, ct, ev, idx)


def _mm2(x, w, w2, bm):
    M, K = x.shape
    P = w.shape[1]
    Q = w2.shape[1]

    def body(x_r, w_r, w2_r, o_r, s_r):
        o = jnp.dot(x_r[...], w_r[...], preferred_element_type=jnp.float32)
        o_r[...] = o
        s_r[...] = jnp.dot(o, w2_r[...], preferred_element_type=jnp.float32)

    return pl.pallas_call(
        body,
        grid=(M // bm,),
        in_specs=[
            pl.BlockSpec((bm, K), lambda i: (i, 0)),
            pl.BlockSpec((K, P), lambda i: (0, 0)),
            pl.BlockSpec((P, Q), lambda i: (0, 0)),
        ],
        out_specs=[
            pl.BlockSpec((bm, P), lambda i: (i, 0)),
            pl.BlockSpec((bm, Q), lambda i: (i, 0)),
        ],
        out_shape=[
            jax.ShapeDtypeStruct((M, P), jnp.float32),
            jax.ShapeDtypeStruct((M, Q), jnp.float32),
        ],
    )(x, w, w2)


def _elu(x):
    return jnp.where(x > 0, x, jnp.exp(jnp.minimum(x, 0.0)) - 1.0)


def _tc_mid(acc1, tc1o, w2cat, a2ocat, bm):
    M = N_PAD

    def body(a_r, t_r, w_r, a2_r, o_r, s_r):
        a = a_r[...]
        t = t_r[...]
        hs = []
        for h in range(2):
            q1 = t[:, 256 * h:256 * h + 128]
            rs = a[h, :, 128] + a[h, :, 129]
            rsafe = jnp.where(rs == 0.0, 1e-12, rs)
            hh = (q1 * rs[:, None] + a[h, :, 0:128]) / rsafe[:, None]
            hs.append(_elu(hh))
        x2 = jnp.concatenate(hs, axis=1)
        o = jnp.dot(x2, w_r[...], preferred_element_type=jnp.float32)
        o_r[...] = o
        s_r[...] = jnp.dot(o, a2_r[...], preferred_element_type=jnp.float32)

    return pl.pallas_call(
        body,
        grid=(M // bm,),
        in_specs=[
            pl.BlockSpec((2, bm, W), lambda i: (0, i, 0)),
            pl.BlockSpec((bm, 512), lambda i: (i, 0)),
            pl.BlockSpec((256, 1024), lambda i: (0, 0)),
            pl.BlockSpec((1024, 128), lambda i: (0, 0)),
        ],
        out_specs=[
            pl.BlockSpec((bm, 1024), lambda i: (i, 0)),
            pl.BlockSpec((bm, 128), lambda i: (i, 0)),
        ],
        out_shape=[
            jax.ShapeDtypeStruct((M, 1024), jnp.float32),
            jax.ShapeDtypeStruct((M, 128), jnp.float32),
        ],
    )(acc1, tc1o, w2cat, a2ocat)


def _tc_final(acc2, tc2o, ent_pad, w_ent, bm):
    M = N_PAD

    def body(a_r, t_r, e_r, w_r, o_r):
        a = a_r[...]
        t = t_r[...]
        q1 = t[:, 0:256]
        q1n = t[:, 256:512]
        rs_r = a[0, :, 128]
        rs_n = a[0, :, 129]
        rst = rs_r + rs_n
        rsafe = jnp.where(rst == 0.0, 1e-12, rst)
        v = jnp.concatenate([a[0, :, 0:128], a[1, :, 0:128]], axis=1)
        h2 = (q1 * rs_r[:, None] + q1n * rs_n[:, None] + v) / rsafe[:, None]
        oe2 = _elu(h2)
        maskv = (a[0, :, 130] > 0.0).astype(jnp.float32)
        eu = jnp.dot(e_r[...], w_r[...], preferred_element_type=jnp.float32)
        o = eu + maskv[:, None] * oe2
        nrm = jnp.sqrt(jnp.sum(o * o, axis=1, keepdims=True))
        o_r[...] = o / jnp.maximum(nrm, 1e-12)

    return pl.pallas_call(
        body,
        grid=(M // bm,),
        in_specs=[
            pl.BlockSpec((2, bm, W), lambda i: (0, i, 0)),
            pl.BlockSpec((bm, 1024), lambda i: (i, 0)),
            pl.BlockSpec((bm, 256), lambda i: (i, 0)),
            pl.BlockSpec((256, 256), lambda i: (0, 0)),
        ],
        out_specs=pl.BlockSpec((bm, 256), lambda i: (i, 0)),
        out_shape=jax.ShapeDtypeStruct((M, 256), jnp.float32),
    )(acc2, tc2o, ent_pad, w_ent)


def _tc_rel2(rel_pad, w_gat, ao3t, a2om):
    def body(r_r, wg_r, a3_r, a2_r, c_r, s_r):
        or1 = jnp.dot(r_r[...], wg_r[...], preferred_element_type=jnp.float32)
        c2 = jnp.dot(or1, a3_r[...], preferred_element_type=jnp.float32)
        c_r[...] = c2
        s_r[...] = jnp.dot(c2, a2_r[...], preferred_element_type=jnp.float32)

    return pl.pallas_call(
        body,
        out_shape=[
            jax.ShapeDtypeStruct((R_PAD, 256), jnp.float32),
            jax.ShapeDtypeStruct((R_PAD, 128), jnp.float32),
        ],
    )(rel_pad, w_gat, ao3t, a2om)


def _tc_rel3(rel_pad, w_ent):
    def body(r_r, w_r, o_r):
        o = jnp.dot(r_r[...], w_r[...], preferred_element_type=jnp.float32)
        nrm = jnp.sqrt(jnp.sum(o * o, axis=1, keepdims=True))
        o_r[...] = o / jnp.maximum(nrm, 1e-12)

    return pl.pallas_call(
        body,
        out_shape=jax.ShapeDtypeStruct((R_PAD, 256), jnp.float32),
    )(rel_pad, w_ent)


def kernel(entity_embeddings, relation_embeddings, a_h0, a2_h0, a_h1, a2_h1,
           a_out, a2_out, W_gat, W_entities, edge_list, edge_type,
           batch_inputs, train_indices_nhop):
    f32 = jnp.float32
    ent_pad = jnp.zeros((N_PAD, 256), f32).at[:N_NODES].set(entity_embeddings)
    rel_pad = jnp.zeros((R_PAD, 256), f32).at[:200].set(relation_embeddings)

    src_r = edge_list[0]
    dst_r = edge_list[1]
    s_n = train_indices_nhop[:, 3]
    d_n = train_indices_nhop[:, 0]
    t0_n = train_indices_nhop[:, 1]
    t1_n = train_indices_nhop[:, 2]
    bt2i = batch_inputs[:, 2]
    n_mask = bt2i.shape[0]
    n_e = src_r.shape[0]
    n_n = s_n.shape[0]
    n_dummy = NT * NB * BE - (n_e + n_n + n_mask)
    i32 = jnp.int32

    def cat(a, b, cm, cd):
        return jnp.concatenate([
            a.astype(i32), b.astype(i32),
            jnp.full((n_mask,), cm, i32), jnp.full((n_dummy,), cd, i32)])

    sq = cat(src_r, s_n + N_PAD, JUNK, JUNK)
    dq = cat(dst_r, d_n + N_PAD, 2 * N_PAD - 1, JUNK)
    t0 = cat(edge_type, t0_n, T_DUMMY, T_DUMMY)
    t1 = cat(jnp.full((n_e,), T_DUMMY, i32), t1_n, T_DUMMY, T_DUMMY)
    ssc = jnp.concatenate([src_r.astype(i32), s_n.astype(i32),
                           bt2i.astype(i32), jnp.full((n_dummy,), JUNK, i32)])
    idx = jnp.stack([x.reshape(NT, NB, BE) for x in (sq, dq, t0, t1, ssc)],
                    axis=2)

    def split3(a):
        return a[:, 0:256], a[:, 256:512], a[:, 512:768]

    w_blocks = []
    for a in (a_h0, a_h1):
        a1, a2, a3 = split3(a)
        w_blocks += [(a1 + a3).T, (a2 + a3).T]
    wcat = jnp.concatenate(w_blocks, axis=1)
    a2cat = jnp.zeros((512, 128), f32)
    a2cat = a2cat.at[0:128, 0].set(a2_h0[0]).at[128:256, 1].set(a2_h0[0])
    a2cat = a2cat.at[256:384, 2].set(a2_h1[0]).at[384:512, 3].set(a2_h1[0])
    wrel = jnp.concatenate([split3(a_h0)[2].T, split3(a_h1)[2].T], axis=1)
    a2catr = jnp.zeros((256, 128), f32)
    a2catr = a2catr.at[0:128, 0].set(a2_h0[0]).at[128:256, 1].set(a2_h1[0])

    tc1o, tc1s = _mm2(ent_pad, wcat, a2cat, 512)
    relo, rels = _mm2(rel_pad, wrel, a2catr, R_PAD)

    def dup2(v):
        return jnp.stack([v, v])

    qa1 = jnp.stack([jnp.concatenate([tc1s[:, 0]] * 2),
                     jnp.concatenate([tc1s[:, 2]] * 2)])
    qb1 = jnp.stack([jnp.concatenate([tc1s[:, 1]] * 2),
                     jnp.concatenate([tc1s[:, 3]] * 2)])
    qc1 = jnp.stack([rels[:, 0], rels[:, 1]])
    e1 = _sc_logits(qa1, qb1, qc1, idx)

    ar = jnp.arange(2 * N_PAD)
    regf = (ar < N_NODES).astype(f32)
    nhf = ((ar >= N_PAD) & (ar < N_PAD + N_NODES)).astype(f32)
    mcols = jnp.zeros((2 * N_PAD, 16), f32).at[:, 0].set(regf).at[:, 1].set(nhf)
    zc16 = jnp.zeros((R_PAD, 16), f32)

    def btab(q2):
        return jnp.concatenate(
            [jnp.concatenate([q2, q2], axis=0), mcols], axis=1)

    bt1 = jnp.stack([btab(tc1o[:, 128:256]), btab(tc1o[:, 384:512])])
    ct1 = jnp.stack(
        [jnp.concatenate([relo[:, 0:128], zc16], axis=1),
         jnp.concatenate([relo[:, 128:256], zc16], axis=1)])
    acc1 = _sc_scatter(bt1, ct1, e1, idx)

    ao1, ao2, ao3 = split3(a_out)
    w2cat = jnp.concatenate([ao1.T, (ao1 + ao3).T, ao2.T, (ao2 + ao3).T],
                            axis=1)
    a2ocat = jnp.zeros((1024, 128), f32)
    for kq in range(4):
        a2ocat = a2ocat.at[256 * kq:256 * (kq + 1), kq].set(a2_out[0])
    tc2o, tc2s = _tc_mid(acc1, tc1o, w2cat, a2ocat, 512)

    a2om = jnp.zeros((256, 128), f32).at[:, 0].set(a2_out[0])
    relo2, rels2 = _tc_rel2(rel_pad, W_gat, ao3.T, a2om)

    qa2 = dup2(jnp.concatenate([tc2s[:, 0], tc2s[:, 1]]))
    qb2 = dup2(jnp.concatenate([tc2s[:, 2], tc2s[:, 3]]))
    qc2 = dup2(rels2[:, 0])
    e2 = _sc_logits(qa2, qb2, qc2, idx)

    maskrow = jnp.zeros((W,), f32).at[130].set(1.0)
    q2l2 = tc2o[:, 512:768]
    q2nl2 = tc2o[:, 768:1024]

    def btab2(k):
        b = jnp.concatenate(
            [jnp.concatenate([q2l2[:, 128 * k:128 * (k + 1)],
                              q2nl2[:, 128 * k:128 * (k + 1)]], axis=0),
             mcols], axis=1)
        return b.at[2 * N_PAD - 1].set(maskrow)

    bt2 = jnp.stack([btab2(0), btab2(1)])
    ct2 = jnp.stack(
        [jnp.concatenate([relo2[:, 0:128], zc16], axis=1),
         jnp.concatenate([relo2[:, 128:256], zc16], axis=1)])
    acc2 = _sc_scatter(bt2, ct2, e2, idx)

    out_ent = _tc_final(acc2, tc2o, ent_pad, W_entities, 512)
    out_rel = _tc_rel3(rel_pad, W_entities)
    return (out_ent[:N_NODES], out_rel[:200])


# speedup vs baseline: 1.0164x; 1.0164x over previous
import functools
import jax
import jax.numpy as jnp
from jax import lax
from jax.experimental import pallas as pl
from jax.experimental.pallas import tpu as pltpu
from jax.experimental.pallas import tpu_sc as plsc

N_NODES = 10000
N_PAD = 10240
JUNK = N_PAD - 1
R_PAD = 208
T_DUMMY = 200
W = 144
NT = 16
BE = 32
NB = 408
CH = 8
ALPHA = 0.2


def _sc_logits(qa, qb, qc, idx):
    mesh = plsc.VectorSubcoreMesh(core_axis_name="c", subcore_axis_name="s")

    @functools.partial(
        pl.kernel,
        out_type=jax.ShapeDtypeStruct((2, NT, NB, BE), jnp.float32),
        mesh=mesh,
        compiler_params=pltpu.CompilerParams(
            use_tc_tiling_on_sc=False, needs_layout_passes=False),
        scratch_types=dict(
            qa_t=pltpu.VMEM((2 * N_PAD,), jnp.float32),
            qb_t=pltpu.VMEM((2 * N_PAD,), jnp.float32),
            qc_t=pltpu.VMEM((R_PAD,), jnp.float32),
            ib=pltpu.VMEM((CH, 4, BE), jnp.int32),
            eb=pltpu.VMEM((CH, BE), jnp.float32),
        ),
    )
    def k(qa_h, qb_h, qc_h, idx_h, e_h, qa_t, qb_t, qc_t, ib, eb):
        c = lax.axis_index("c")
        s = lax.axis_index("s")
        pltpu.sync_copy(qa_h.at[c], qa_t)
        pltpu.sync_copy(qb_h.at[c], qb_t)
        pltpu.sync_copy(qc_h.at[c], qc_t)

        def chunk(g, _):
            pltpu.sync_copy(idx_h.at[s, pl.ds(g * CH, CH)], ib)

            def batch(b, _):
                def grp(h, _):
                    sl = pl.ds(h * 16, 16)
                    sv = (plsc.load_gather(qa_t, [ib[b, 0, sl]])
                          + plsc.load_gather(qb_t, [ib[b, 1, sl]])
                          + plsc.load_gather(qc_t, [ib[b, 2, sl]])
                          + plsc.load_gather(qc_t, [ib[b, 3, sl]]))
                    eb[b, sl] = jnp.exp(-jnp.maximum(sv, ALPHA * sv))
                    return 0

                return lax.fori_loop(0, BE // 16, grp, 0)

            lax.fori_loop(0, CH, batch, 0)
            pltpu.sync_copy(eb, e_h.at[c, s, pl.ds(g * CH, CH)])
            return 0

        lax.fori_loop(0, NB // CH, chunk, 0)

    return k(qa, qb, qc, idx)


def _sc_scatter(gt, ev, gidx, sidx):
    mesh = plsc.VectorSubcoreMesh(core_axis_name="c", subcore_axis_name="s")
    rows_per = N_PAD // NT
    GE = 3 * BE

    @functools.partial(
        pl.kernel,
        out_type=jax.ShapeDtypeStruct((2, N_PAD, W), jnp.float32),
        mesh=mesh,
        compiler_params=pltpu.CompilerParams(
            use_tc_tiling_on_sc=False, needs_layout_passes=False),
        scratch_types=dict(
            gib=pltpu.VMEM((CH, GE), jnp.int32),
            sib=pltpu.VMEM((CH, BE), jnp.int32),
            eb=pltpu.VMEM((CH, BE), jnp.float32),
            gb=pltpu.VMEM((2, GE, W), jnp.float32),
            zb=pltpu.VMEM((BE, W), jnp.float32),
            acc=pltpu.VMEM_SHARED((N_PAD, W), jnp.float32),
            gsem=pltpu.SemaphoreType.DMA((2,)),
        ),
    )
    def k(gt_h, ev_h, gidx_h, sidx_h, out_h, gib, sib, eb, gb, zb, acc, gsem):
        c = lax.axis_index("c")
        s = lax.axis_index("s")

        def zi(i, _):
            for j in range(W // 16):
                zb[i, pl.ds(j * 16, 16)] = jnp.zeros((16,), jnp.float32)
            return 0

        lax.fori_loop(0, BE, zi, 0)
        for r in range(rows_per // BE):
            pltpu.sync_copy(zb, acc.at[pl.ds(s * rows_per + r * BE, BE)])
        plsc.subcore_barrier()

        def fire(b, p):
            pltpu.async_copy(gt_h.at[c].at[gib.at[b]], gb.at[p], gsem.at[p])

        def drain(b, p):
            pltpu.make_async_copy(gt_h.at[c].at[gib.at[b]], gb.at[p],
                                  gsem.at[p]).wait()

        def chunk(g, _):
            pltpu.sync_copy(gidx_h.at[s, pl.ds(g * CH, CH)], gib)
            pltpu.sync_copy(sidx_h.at[s, pl.ds(g * CH, CH)], sib)
            pltpu.sync_copy(ev_h.at[c, s, pl.ds(g * CH, CH)], eb)
            fire(0, 0)

            def batch(b, _):
                p = lax.rem(b, 2)

                @pl.when(b + 1 < CH)
                def _():
                    fire(b + 1, 1 - p)

                drain(b, p)

                def grp(h, _):
                    e16 = eb[b, pl.ds(h * 16, 16)]
                    for j in range(16):
                        i = h * 16 + j
                        e_i = e16[j]
                        for kk in range(W // 16):
                            sl = pl.ds(kk * 16, 16)
                            if kk == 8:
                                gb[p, i, sl] = gb[p, i, sl] * e_i
                            else:
                                gb[p, i, sl] = (gb[p, i, sl]
                                                + gb[p, BE + i, sl]
                                                + gb[p, 2 * BE + i, sl]) * e_i
                    return 0

                lax.fori_loop(0, BE // 16, grp, 0)
                pltpu.sync_copy(gb.at[p, pl.ds(0, BE)], acc.at[sib.at[b]],
                                add=True)
                return 0

            lax.fori_loop(0, CH, batch, 0)
            return 0

        lax.fori_loop(0, NB // CH, chunk, 0)
        plsc.subcore_barrier()
        pltpu.sync_copy(acc.at[pl.ds(s * rows_per, rows_per)],
                        out_h.at[c].at[pl.ds(s * rows_per, rows_per)])

    return k(gt, ev, gidx, sidx)


def _mm2(x, w, w2, bm):
    M, K = x.shape
    P = w.shape[1]
    Q = w2.shape[1]

    def body(x_r, w_r, w2_r, o_r, s_r):
        o = jnp.dot(x_r[...], w_r[...], preferred_element_type=jnp.float32)
        o_r[...] = o
        s_r[...] = jnp.dot(o, w2_r[...], preferred_element_type=jnp.float32)

    return pl.pallas_call(
        body,
        grid=(M // bm,),
        in_specs=[
            pl.BlockSpec((bm, K), lambda i: (i, 0)),
            pl.BlockSpec((K, P), lambda i: (0, 0)),
            pl.BlockSpec((P, Q), lambda i: (0, 0)),
        ],
        out_specs=[
            pl.BlockSpec((bm, P), lambda i: (i, 0)),
            pl.BlockSpec((bm, Q), lambda i: (i, 0)),
        ],
        out_shape=[
            jax.ShapeDtypeStruct((M, P), jnp.float32),
            jax.ShapeDtypeStruct((M, Q), jnp.float32),
        ],
    )(x, w, w2)


def _elu(x):
    return jnp.where(x > 0, x, jnp.exp(jnp.minimum(x, 0.0)) - 1.0)


def _tc_mid(acc1, tc1o, w2cat, a2ocat, bm):
    M = N_PAD

    def body(a_r, t_r, w_r, a2_r, o_r, s_r):
        a = a_r[...]
        t = t_r[...]
        hs = []
        for h in range(2):
            q1 = t[:, 256 * h:256 * h + 128]
            rs = a[h, :, 128] + a[h, :, 129]
            rsafe = jnp.where(rs == 0.0, 1e-12, rs)
            hh = (q1 * rs[:, None] + a[h, :, 0:128]) / rsafe[:, None]
            hs.append(_elu(hh))
        x2 = jnp.concatenate(hs, axis=1)
        o = jnp.dot(x2, w_r[...], preferred_element_type=jnp.float32)
        o_r[...] = o
        s_r[...] = jnp.dot(o, a2_r[...], preferred_element_type=jnp.float32)

    return pl.pallas_call(
        body,
        grid=(M // bm,),
        in_specs=[
            pl.BlockSpec((2, bm, W), lambda i: (0, i, 0)),
            pl.BlockSpec((bm, 512), lambda i: (i, 0)),
            pl.BlockSpec((256, 1024), lambda i: (0, 0)),
            pl.BlockSpec((1024, 128), lambda i: (0, 0)),
        ],
        out_specs=[
            pl.BlockSpec((bm, 1024), lambda i: (i, 0)),
            pl.BlockSpec((bm, 128), lambda i: (i, 0)),
        ],
        out_shape=[
            jax.ShapeDtypeStruct((M, 1024), jnp.float32),
            jax.ShapeDtypeStruct((M, 128), jnp.float32),
        ],
    )(acc1, tc1o, w2cat, a2ocat)


def _tc_final(acc2, tc2o, ent_pad, w_ent, bm):
    M = N_PAD

    def body(a_r, t_r, e_r, w_r, o_r):
        a = a_r[...]
        t = t_r[...]
        q1 = t[:, 0:256]
        q1n = t[:, 256:512]
        rs_r = a[0, :, 128]
        rs_n = a[0, :, 129]
        rst = rs_r + rs_n
        rsafe = jnp.where(rst == 0.0, 1e-12, rst)
        v = jnp.concatenate([a[0, :, 0:128], a[1, :, 0:128]], axis=1)
        h2 = (q1 * rs_r[:, None] + q1n * rs_n[:, None] + v) / rsafe[:, None]
        oe2 = _elu(h2)
        maskv = (a[0, :, 130] > 0.0).astype(jnp.float32)
        eu = jnp.dot(e_r[...], w_r[...], preferred_element_type=jnp.float32)
        o = eu + maskv[:, None] * oe2
        nrm = jnp.sqrt(jnp.sum(o * o, axis=1, keepdims=True))
        o_r[...] = o / jnp.maximum(nrm, 1e-12)

    return pl.pallas_call(
        body,
        grid=(M // bm,),
        in_specs=[
            pl.BlockSpec((2, bm, W), lambda i: (0, i, 0)),
            pl.BlockSpec((bm, 1024), lambda i: (i, 0)),
            pl.BlockSpec((bm, 256), lambda i: (i, 0)),
            pl.BlockSpec((256, 256), lambda i: (0, 0)),
        ],
        out_specs=pl.BlockSpec((bm, 256), lambda i: (i, 0)),
        out_shape=jax.ShapeDtypeStruct((M, 256), jnp.float32),
    )(acc2, tc2o, ent_pad, w_ent)


def _tc_rel2(rel_pad, w_gat, ao3t, a2om):
    def body(r_r, wg_r, a3_r, a2_r, c_r, s_r):
        or1 = jnp.dot(r_r[...], wg_r[...], preferred_element_type=jnp.float32)
        c2 = jnp.dot(or1, a3_r[...], preferred_element_type=jnp.float32)
        c_r[...] = c2
        s_r[...] = jnp.dot(c2, a2_r[...], preferred_element_type=jnp.float32)

    return pl.pallas_call(
        body,
        out_shape=[
            jax.ShapeDtypeStruct((R_PAD, 256), jnp.float32),
            jax.ShapeDtypeStruct((R_PAD, 128), jnp.float32),
        ],
    )(rel_pad, w_gat, ao3t, a2om)


def _tc_rel3(rel_pad, w_ent):
    def body(r_r, w_r, o_r):
        o = jnp.dot(r_r[...], w_r[...], preferred_element_type=jnp.float32)
        nrm = jnp.sqrt(jnp.sum(o * o, axis=1, keepdims=True))
        o_r[...] = o / jnp.maximum(nrm, 1e-12)

    return pl.pallas_call(
        body,
        out_shape=jax.ShapeDtypeStruct((R_PAD, 256), jnp.float32),
    )(rel_pad, w_ent)


def kernel(entity_embeddings, relation_embeddings, a_h0, a2_h0, a_h1, a2_h1,
           a_out, a2_out, W_gat, W_entities, edge_list, edge_type,
           batch_inputs, train_indices_nhop):
    f32 = jnp.float32
    ent_pad = jnp.zeros((N_PAD, 256), f32).at[:N_NODES].set(entity_embeddings)
    rel_pad = jnp.zeros((R_PAD, 256), f32).at[:200].set(relation_embeddings)

    src_r = edge_list[0]
    dst_r = edge_list[1]
    s_n = train_indices_nhop[:, 3]
    d_n = train_indices_nhop[:, 0]
    t0_n = train_indices_nhop[:, 1]
    t1_n = train_indices_nhop[:, 2]
    bt2i = batch_inputs[:, 2]
    n_mask = bt2i.shape[0]
    n_e = src_r.shape[0]
    n_n = s_n.shape[0]
    n_dummy = NT * NB * BE - (n_e + n_n + n_mask)
    i32 = jnp.int32

    def cat(a, b, cm, cd):
        return jnp.concatenate([
            a.astype(i32), b.astype(i32),
            jnp.full((n_mask,), cm, i32), jnp.full((n_dummy,), cd, i32)])

    sq = cat(src_r, s_n + N_PAD, JUNK, JUNK)
    dq = cat(dst_r, d_n + N_PAD, 2 * N_PAD - 1, JUNK)
    t0 = cat(edge_type, t0_n, T_DUMMY, T_DUMMY)
    t1 = cat(jnp.full((n_e,), T_DUMMY, i32), t1_n, T_DUMMY, T_DUMMY)
    ssc = jnp.concatenate([src_r.astype(i32), s_n.astype(i32),
                           bt2i.astype(i32), jnp.full((n_dummy,), JUNK, i32)])
    idx = jnp.stack([x.reshape(NT, NB, BE) for x in (sq, dq, t0, t1)],
                    axis=2)
    OFF = 2 * N_PAD
    gidx = jnp.concatenate([dq.reshape(NT, NB, BE),
                            t0.reshape(NT, NB, BE) + OFF,
                            t1.reshape(NT, NB, BE) + OFF], axis=2)
    sidx = ssc.reshape(NT, NB, BE)

    def split3(a):
        return a[:, 0:256], a[:, 256:512], a[:, 512:768]

    w_blocks = []
    for a in (a_h0, a_h1):
        a1, a2, a3 = split3(a)
        w_blocks += [(a1 + a3).T, (a2 + a3).T]
    wcat = jnp.concatenate(w_blocks, axis=1)
    a2cat = jnp.zeros((512, 128), f32)
    a2cat = a2cat.at[0:128, 0].set(a2_h0[0]).at[128:256, 1].set(a2_h0[0])
    a2cat = a2cat.at[256:384, 2].set(a2_h1[0]).at[384:512, 3].set(a2_h1[0])
    wrel = jnp.concatenate([split3(a_h0)[2].T, split3(a_h1)[2].T], axis=1)
    a2catr = jnp.zeros((256, 128), f32)
    a2catr = a2catr.at[0:128, 0].set(a2_h0[0]).at[128:256, 1].set(a2_h1[0])

    tc1o, tc1s = _mm2(ent_pad, wcat, a2cat, 512)
    relo, rels = _mm2(rel_pad, wrel, a2catr, R_PAD)

    def dup2(v):
        return jnp.stack([v, v])

    qa1 = jnp.stack([jnp.concatenate([tc1s[:, 0]] * 2),
                     jnp.concatenate([tc1s[:, 2]] * 2)])
    qb1 = jnp.stack([jnp.concatenate([tc1s[:, 1]] * 2),
                     jnp.concatenate([tc1s[:, 3]] * 2)])
    qc1 = jnp.stack([rels[:, 0], rels[:, 1]])
    e1 = _sc_logits(qa1, qb1, qc1, idx)

    ar = jnp.arange(2 * N_PAD)
    regf = (ar < N_NODES).astype(f32)
    nhf = ((ar >= N_PAD) & (ar < N_PAD + N_NODES)).astype(f32)
    mcols = jnp.zeros((2 * N_PAD, 16), f32).at[:, 0].set(regf).at[:, 1].set(nhf)
    zc16 = jnp.zeros((R_PAD, 16), f32)

    def btab(q2):
        return jnp.concatenate(
            [jnp.concatenate([q2, q2], axis=0), mcols], axis=1)

    bt1 = jnp.stack([btab(tc1o[:, 128:256]), btab(tc1o[:, 384:512])])
    ct1 = jnp.stack(
        [jnp.concatenate([relo[:, 0:128], zc16], axis=1),
         jnp.concatenate([relo[:, 128:256], zc16], axis=1)])
    gt1 = jnp.concatenate([bt1, ct1], axis=1)
    acc1 = _sc_scatter(gt1, e1, gidx, sidx)

    ao1, ao2, ao3 = split3(a_out)
    w2cat = jnp.concatenate([ao1.T, (ao1 + ao3).T, ao2.T, (ao2 + ao3).T],
                            axis=1)
    a2ocat = jnp.zeros((1024, 128), f32)
    for kq in range(4):
        a2ocat = a2ocat.at[256 * kq:256 * (kq + 1), kq].set(a2_out[0])
    tc2o, tc2s = _tc_mid(acc1, tc1o, w2cat, a2ocat, 512)

    a2om = jnp.zeros((256, 128), f32).at[:, 0].set(a2_out[0])
    relo2, rels2 = _tc_rel2(rel_pad, W_gat, ao3.T, a2om)

    qa2 = dup2(jnp.concatenate([tc2s[:, 0], tc2s[:, 1]]))
    qb2 = dup2(jnp.concatenate([tc2s[:, 2], tc2s[:, 3]]))
    qc2 = dup2(rels2[:, 0])
    e2 = _sc_logits(qa2, qb2, qc2, idx)

    maskrow = jnp.zeros((W,), f32).at[130].set(1.0)
    q2l2 = tc2o[:, 512:768]
    q2nl2 = tc2o[:, 768:1024]

    def btab2(k):
        b = jnp.concatenate(
            [jnp.concatenate([q2l2[:, 128 * k:128 * (k + 1)],
                              q2nl2[:, 128 * k:128 * (k + 1)]], axis=0),
             mcols], axis=1)
        return b.at[2 * N_PAD - 1].set(maskrow)

    bt2 = jnp.stack([btab2(0), btab2(1)])
    ct2 = jnp.stack(
        [jnp.concatenate([relo2[:, 0:128], zc16], axis=1),
         jnp.concatenate([relo2[:, 128:256], zc16], axis=1)])
    gt2 = jnp.concatenate([bt2, ct2], axis=1)
    acc2 = _sc_scatter(gt2, e2, gidx, sidx)

    out_ent = _tc_final(acc2, tc2o, ent_pad, W_entities, 512)
    out_rel = _tc_rel3(rel_pad, W_entities)
    return (out_ent[:N_NODES], out_rel[:200])


# speedup vs baseline: 3.8665x; 3.8042x over previous
import functools
import jax
import jax.numpy as jnp
from jax import lax
from jax.experimental import pallas as pl
from jax.experimental.pallas import tpu as pltpu
from jax.experimental.pallas import tpu_sc as plsc

N_NODES = 10000
N_PAD = 10240
JUNK = N_PAD - 1
R_PAD = 208
T_DUMMY = 200
W = 144
NT = 16
BE = 32
NB = 416
REG_B = 320
CH = 8
ALPHA = 0.2


def _sc_logits(qa, qb, qc, idx):
    mesh = plsc.VectorSubcoreMesh(core_axis_name="c", subcore_axis_name="s")

    @functools.partial(
        pl.kernel,
        out_type=jax.ShapeDtypeStruct((2, NT, NB, BE), jnp.float32),
        mesh=mesh,
        compiler_params=pltpu.CompilerParams(
            use_tc_tiling_on_sc=False, needs_layout_passes=False),
        scratch_types=dict(
            qa_t=pltpu.VMEM((2 * N_PAD,), jnp.float32),
            qb_t=pltpu.VMEM((2 * N_PAD,), jnp.float32),
            qc_t=pltpu.VMEM((R_PAD,), jnp.float32),
            ib=pltpu.VMEM((CH, 4, BE), jnp.int32),
            eb=pltpu.VMEM((CH, BE), jnp.float32),
        ),
    )
    def k(qa_h, qb_h, qc_h, idx_h, e_h, qa_t, qb_t, qc_t, ib, eb):
        c = lax.axis_index("c")
        s = lax.axis_index("s")
        pltpu.sync_copy(qa_h.at[c], qa_t)
        pltpu.sync_copy(qb_h.at[c], qb_t)
        pltpu.sync_copy(qc_h.at[c], qc_t)

        def chunk(g, _):
            pltpu.sync_copy(idx_h.at[s, pl.ds(g * CH, CH)], ib)

            def batch(b, _):
                def grp(h, _):
                    sl = pl.ds(h * 16, 16)
                    sv = (plsc.load_gather(qa_t, [ib[b, 0, sl]])
                          + plsc.load_gather(qb_t, [ib[b, 1, sl]])
                          + plsc.load_gather(qc_t, [ib[b, 2, sl]])
                          + plsc.load_gather(qc_t, [ib[b, 3, sl]]))
                    eb[b, sl] = jnp.exp(-jnp.maximum(sv, ALPHA * sv))
                    return 0

                return lax.fori_loop(0, BE // 16, grp, 0)

            lax.fori_loop(0, CH, batch, 0)
            pltpu.sync_copy(eb, e_h.at[c, s, pl.ds(g * CH, CH)])
            return 0

        lax.fori_loop(0, NB // CH, chunk, 0)

    return k(qa, qb, qc, idx)


def _sc_scatter(gt, ev, gidx, sidx):
    mesh = plsc.VectorSubcoreMesh(core_axis_name="c", subcore_axis_name="s")
    rows_per = N_PAD // NT
    GE = 3 * BE

    @functools.partial(
        pl.kernel,
        out_type=jax.ShapeDtypeStruct((2, N_PAD, W), jnp.float32),
        mesh=mesh,
        compiler_params=pltpu.CompilerParams(
            use_tc_tiling_on_sc=False, needs_layout_passes=False),
        scratch_types=dict(
            gib=pltpu.VMEM((CH, GE), jnp.int32),
            sib=pltpu.VMEM((CH, BE), jnp.int32),
            eb=pltpu.VMEM((CH, BE), jnp.float32),
            gb=pltpu.VMEM((2, GE, W), jnp.float32),
            zb=pltpu.VMEM((BE, W), jnp.float32),
            acc=pltpu.VMEM_SHARED((N_PAD, W), jnp.float32),
            gsem=pltpu.SemaphoreType.DMA((2,)),
        ),
    )
    def k(gt_h, ev_h, gidx_h, sidx_h, out_h, gib, sib, eb, gb, zb, acc, gsem):
        c = lax.axis_index("c")
        s = lax.axis_index("s")

        def zi(i, _):
            for j in range(W // 16):
                zb[i, pl.ds(j * 16, 16)] = jnp.zeros((16,), jnp.float32)
            return 0

        lax.fori_loop(0, BE, zi, 0)
        for r in range(rows_per // BE):
            pltpu.sync_copy(zb, acc.at[pl.ds(s * rows_per + r * BE, BE)])
        plsc.subcore_barrier()

        def make_chunk(with_c1):
            nrow = 3 * BE if with_c1 else 2 * BE

            def fire(b, p):
                pltpu.async_copy(gt_h.at[c].at[gib.at[b, pl.ds(0, nrow)]],
                                 gb.at[p, pl.ds(0, nrow)], gsem.at[p])

            def drain(b, p):
                pltpu.make_async_copy(gt_h.at[c].at[gib.at[b, pl.ds(0, nrow)]],
                                      gb.at[p, pl.ds(0, nrow)],
                                      gsem.at[p]).wait()

            def chunk(g, _):
                pltpu.sync_copy(gidx_h.at[s, pl.ds(g * CH, CH)], gib)
                pltpu.sync_copy(sidx_h.at[s, pl.ds(g * CH, CH)], sib)
                pltpu.sync_copy(ev_h.at[c, s, pl.ds(g * CH, CH)], eb)
                fire(0, 0)

                def batch(b, _):
                    p = lax.rem(b, 2)

                    @pl.when(b + 1 < CH)
                    def _():
                        fire(b + 1, 1 - p)

                    drain(b, p)

                    def grp(h, _):
                        e16 = eb[b, pl.ds(h * 16, 16)]
                        for j in range(16):
                            i = h * 16 + j
                            e_i = e16[j]
                            for kk in range(W // 16):
                                sl = pl.ds(kk * 16, 16)
                                if kk == 8:
                                    gb[p, i, sl] = gb[p, i, sl] * e_i
                                elif with_c1:
                                    gb[p, i, sl] = (gb[p, i, sl]
                                                    + gb[p, BE + i, sl]
                                                    + gb[p, 2 * BE + i, sl]
                                                    ) * e_i
                                else:
                                    gb[p, i, sl] = (gb[p, i, sl]
                                                    + gb[p, BE + i, sl]
                                                    ) * e_i
                        return 0

                    lax.fori_loop(0, BE // 16, grp, 0)
                    pltpu.sync_copy(gb.at[p, pl.ds(0, BE)],
                                    acc.at[sib.at[b]], add=True)
                    return 0

                lax.fori_loop(0, CH, batch, 0)
                return 0

            return chunk

        lax.fori_loop(0, REG_B // CH, make_chunk(False), 0)
        lax.fori_loop(REG_B // CH, NB // CH, make_chunk(True), 0)
        plsc.subcore_barrier()
        pltpu.sync_copy(acc.at[pl.ds(s * rows_per, rows_per)],
                        out_h.at[c].at[pl.ds(s * rows_per, rows_per)])

    return k(gt, ev, gidx, sidx)


def _mm2(x, w, w2, bm):
    M, K = x.shape
    P = w.shape[1]
    Q = w2.shape[1]

    def body(x_r, w_r, w2_r, o_r, s_r):
        o = jnp.dot(x_r[...], w_r[...], preferred_element_type=jnp.float32)
        o_r[...] = o
        s_r[...] = jnp.dot(o, w2_r[...], preferred_element_type=jnp.float32)

    return pl.pallas_call(
        body,
        grid=(M // bm,),
        in_specs=[
            pl.BlockSpec((bm, K), lambda i: (i, 0)),
            pl.BlockSpec((K, P), lambda i: (0, 0)),
            pl.BlockSpec((P, Q), lambda i: (0, 0)),
        ],
        out_specs=[
            pl.BlockSpec((bm, P), lambda i: (i, 0)),
            pl.BlockSpec((bm, Q), lambda i: (i, 0)),
        ],
        out_shape=[
            jax.ShapeDtypeStruct((M, P), jnp.float32),
            jax.ShapeDtypeStruct((M, Q), jnp.float32),
        ],
    )(x, w, w2)


def _elu(x):
    return jnp.where(x > 0, x, jnp.exp(jnp.minimum(x, 0.0)) - 1.0)


def _tc_mid(acc1, tc1o, w2cat, a2ocat, bm):
    M = N_PAD

    def body(a_r, t_r, w_r, a2_r, o_r, s_r):
        a = a_r[...]
        t = t_r[...]
        hs = []
        for h in range(2):
            q1 = t[:, 256 * h:256 * h + 128]
            rs = a[h, :, 128] + a[h, :, 129]
            rsafe = jnp.where(rs == 0.0, 1e-12, rs)
            hh = (q1 * rs[:, None] + a[h, :, 0:128]) / rsafe[:, None]
            hs.append(_elu(hh))
        x2 = jnp.concatenate(hs, axis=1)
        o = jnp.dot(x2, w_r[...], preferred_element_type=jnp.float32)
        o_r[...] = o
        s_r[...] = jnp.dot(o, a2_r[...], preferred_element_type=jnp.float32)

    return pl.pallas_call(
        body,
        grid=(M // bm,),
        in_specs=[
            pl.BlockSpec((2, bm, W), lambda i: (0, i, 0)),
            pl.BlockSpec((bm, 512), lambda i: (i, 0)),
            pl.BlockSpec((256, 1024), lambda i: (0, 0)),
            pl.BlockSpec((1024, 128), lambda i: (0, 0)),
        ],
        out_specs=[
            pl.BlockSpec((bm, 1024), lambda i: (i, 0)),
            pl.BlockSpec((bm, 128), lambda i: (i, 0)),
        ],
        out_shape=[
            jax.ShapeDtypeStruct((M, 1024), jnp.float32),
            jax.ShapeDtypeStruct((M, 128), jnp.float32),
        ],
    )(acc1, tc1o, w2cat, a2ocat)


def _tc_final(acc2, tc2o, ent_pad, w_ent, bm):
    M = N_PAD

    def body(a_r, t_r, e_r, w_r, o_r):
        a = a_r[...]
        t = t_r[...]
        q1 = t[:, 0:256]
        q1n = t[:, 256:512]
        rs_r = a[0, :, 128]
        rs_n = a[0, :, 129]
        rst = rs_r + rs_n
        rsafe = jnp.where(rst == 0.0, 1e-12, rst)
        v = jnp.concatenate([a[0, :, 0:128], a[1, :, 0:128]], axis=1)
        h2 = (q1 * rs_r[:, None] + q1n * rs_n[:, None] + v) / rsafe[:, None]
        oe2 = _elu(h2)
        maskv = (a[0, :, 130] > 0.0).astype(jnp.float32)
        eu = jnp.dot(e_r[...], w_r[...], preferred_element_type=jnp.float32)
        o = eu + maskv[:, None] * oe2
        nrm = jnp.sqrt(jnp.sum(o * o, axis=1, keepdims=True))
        o_r[...] = o / jnp.maximum(nrm, 1e-12)

    return pl.pallas_call(
        body,
        grid=(M // bm,),
        in_specs=[
            pl.BlockSpec((2, bm, W), lambda i: (0, i, 0)),
            pl.BlockSpec((bm, 1024), lambda i: (i, 0)),
            pl.BlockSpec((bm, 256), lambda i: (i, 0)),
            pl.BlockSpec((256, 256), lambda i: (0, 0)),
        ],
        out_specs=pl.BlockSpec((bm, 256), lambda i: (i, 0)),
        out_shape=jax.ShapeDtypeStruct((M, 256), jnp.float32),
    )(acc2, tc2o, ent_pad, w_ent)


def _tc_rel2(rel_pad, w_gat, ao3t, a2om):
    def body(r_r, wg_r, a3_r, a2_r, c_r, s_r):
        or1 = jnp.dot(r_r[...], wg_r[...], preferred_element_type=jnp.float32)
        c2 = jnp.dot(or1, a3_r[...], preferred_element_type=jnp.float32)
        c_r[...] = c2
        s_r[...] = jnp.dot(c2, a2_r[...], preferred_element_type=jnp.float32)

    return pl.pallas_call(
        body,
        out_shape=[
            jax.ShapeDtypeStruct((R_PAD, 256), jnp.float32),
            jax.ShapeDtypeStruct((R_PAD, 128), jnp.float32),
        ],
    )(rel_pad, w_gat, ao3t, a2om)


def _tc_rel3(rel_pad, w_ent):
    def body(r_r, w_r, o_r):
        o = jnp.dot(r_r[...], w_r[...], preferred_element_type=jnp.float32)
        nrm = jnp.sqrt(jnp.sum(o * o, axis=1, keepdims=True))
        o_r[...] = o / jnp.maximum(nrm, 1e-12)

    return pl.pallas_call(
        body,
        out_shape=jax.ShapeDtypeStruct((R_PAD, 256), jnp.float32),
    )(rel_pad, w_ent)


def kernel(entity_embeddings, relation_embeddings, a_h0, a2_h0, a_h1, a2_h1,
           a_out, a2_out, W_gat, W_entities, edge_list, edge_type,
           batch_inputs, train_indices_nhop):
    f32 = jnp.float32
    ent_pad = jnp.zeros((N_PAD, 256), f32).at[:N_NODES].set(entity_embeddings)
    rel_pad = jnp.zeros((R_PAD, 256), f32).at[:200].set(relation_embeddings)

    src_r = edge_list[0]
    dst_r = edge_list[1]
    s_n = train_indices_nhop[:, 3]
    d_n = train_indices_nhop[:, 0]
    t0_n = train_indices_nhop[:, 1]
    t1_n = train_indices_nhop[:, 2]
    bt2i = batch_inputs[:, 2]
    n_mask = bt2i.shape[0]
    n_e = src_r.shape[0]
    i32 = jnp.int32
    reg_slots = NT * REG_B * BE
    tail_slots = NT * (NB - REG_B) * BE

    def sect(x_reg, x_tail, fr, ft):
        a = jnp.concatenate([
            x_reg.astype(i32),
            jnp.full((reg_slots - x_reg.shape[0],), fr, i32)]).reshape(NT, -1)
        b = jnp.concatenate([
            x_tail.astype(i32),
            jnp.full((tail_slots - x_tail.shape[0],), ft, i32)]).reshape(NT, -1)
        return jnp.concatenate([a, b], axis=1).reshape(-1)

    sq = sect(src_r, jnp.concatenate([s_n + N_PAD, jnp.full((n_mask,), JUNK, i32)]), JUNK, JUNK)
    dq = sect(dst_r, jnp.concatenate([d_n + N_PAD, jnp.full((n_mask,), 2 * N_PAD - 1, i32)]), JUNK, JUNK)
    t0 = sect(edge_type, jnp.concatenate([t0_n, jnp.full((n_mask,), T_DUMMY, i32)]), T_DUMMY, T_DUMMY)
    t1 = sect(jnp.full((n_e,), T_DUMMY, i32), jnp.concatenate([t1_n, jnp.full((n_mask,), T_DUMMY, i32)]), T_DUMMY, T_DUMMY)
    ssc = sect(src_r, jnp.concatenate([s_n, bt2i]), JUNK, JUNK)
    idx = jnp.stack([x.reshape(NT, NB, BE) for x in (sq, dq, t0, t1)],
                    axis=2)
    OFF = 2 * N_PAD
    gidx = jnp.concatenate([dq.reshape(NT, NB, BE),
                            t0.reshape(NT, NB, BE) + OFF,
                            t1.reshape(NT, NB, BE) + OFF], axis=2)
    sidx = ssc.reshape(NT, NB, BE)

    def split3(a):
        return a[:, 0:256], a[:, 256:512], a[:, 512:768]

    w_blocks = []
    for a in (a_h0, a_h1):
        a1, a2, a3 = split3(a)
        w_blocks += [(a1 + a3).T, (a2 + a3).T]
    wcat = jnp.concatenate(w_blocks, axis=1)
    a2cat = jnp.zeros((512, 128), f32)
    a2cat = a2cat.at[0:128, 0].set(a2_h0[0]).at[128:256, 1].set(a2_h0[0])
    a2cat = a2cat.at[256:384, 2].set(a2_h1[0]).at[384:512, 3].set(a2_h1[0])
    wrel = jnp.concatenate([split3(a_h0)[2].T, split3(a_h1)[2].T], axis=1)
    a2catr = jnp.zeros((256, 128), f32)
    a2catr = a2catr.at[0:128, 0].set(a2_h0[0]).at[128:256, 1].set(a2_h1[0])

    tc1o, tc1s = _mm2(ent_pad, wcat, a2cat, 512)
    relo, rels = _mm2(rel_pad, wrel, a2catr, R_PAD)

    def dup2(v):
        return jnp.stack([v, v])

    qa1 = jnp.stack([jnp.concatenate([tc1s[:, 0]] * 2),
                     jnp.concatenate([tc1s[:, 2]] * 2)])
    qb1 = jnp.stack([jnp.concatenate([tc1s[:, 1]] * 2),
                     jnp.concatenate([tc1s[:, 3]] * 2)])
    qc1 = jnp.stack([rels[:, 0], rels[:, 1]])
    e1 = _sc_logits(qa1, qb1, qc1, idx)

    ar = jnp.arange(2 * N_PAD)
    regf = (ar < N_NODES).astype(f32)
    nhf = ((ar >= N_PAD) & (ar < N_PAD + N_NODES)).astype(f32)
    mcols = jnp.zeros((2 * N_PAD, 16), f32).at[:, 0].set(regf).at[:, 1].set(nhf)
    zc16 = jnp.zeros((R_PAD, 16), f32)

    def btab(q2):
        return jnp.concatenate(
            [jnp.concatenate([q2, q2], axis=0), mcols], axis=1)

    bt1 = jnp.stack([btab(tc1o[:, 128:256]), btab(tc1o[:, 384:512])])
    ct1 = jnp.stack(
        [jnp.concatenate([relo[:, 0:128], zc16], axis=1),
         jnp.concatenate([relo[:, 128:256], zc16], axis=1)])
    gt1 = jnp.concatenate([bt1, ct1], axis=1)
    acc1 = _sc_scatter(gt1, e1, gidx, sidx)

    ao1, ao2, ao3 = split3(a_out)
    w2cat = jnp.concatenate([ao1.T, (ao1 + ao3).T, ao2.T, (ao2 + ao3).T],
                            axis=1)
    a2ocat = jnp.zeros((1024, 128), f32)
    for kq in range(4):
        a2ocat = a2ocat.at[256 * kq:256 * (kq + 1), kq].set(a2_out[0])
    tc2o, tc2s = _tc_mid(acc1, tc1o, w2cat, a2ocat, 512)

    a2om = jnp.zeros((256, 128), f32).at[:, 0].set(a2_out[0])
    relo2, rels2 = _tc_rel2(rel_pad, W_gat, ao3.T, a2om)

    qa2 = dup2(jnp.concatenate([tc2s[:, 0], tc2s[:, 1]]))
    qb2 = dup2(jnp.concatenate([tc2s[:, 2], tc2s[:, 3]]))
    qc2 = dup2(rels2[:, 0])
    e2 = _sc_logits(qa2, qb2, qc2, idx)

    maskrow = jnp.zeros((W,), f32).at[130].set(1.0)
    q2l2 = tc2o[:, 512:768]
    q2nl2 = tc2o[:, 768:1024]

    def btab2(k):
        b = jnp.concatenate(
            [jnp.concatenate([q2l2[:, 128 * k:128 * (k + 1)],
                              q2nl2[:, 128 * k:128 * (k + 1)]], axis=0),
             mcols], axis=1)
        return b.at[2 * N_PAD - 1].set(maskrow)

    bt2 = jnp.stack([btab2(0), btab2(1)])
    ct2 = jnp.stack(
        [jnp.concatenate([relo2[:, 0:128], zc16], axis=1),
         jnp.concatenate([relo2[:, 128:256], zc16], axis=1)])
    gt2 = jnp.concatenate([bt2, ct2], axis=1)
    acc2 = _sc_scatter(gt2, e2, gidx, sidx)

    out_ent = _tc_final(acc2, tc2o, ent_pad, W_entities, 512)
    out_rel = _tc_rel3(rel_pad, W_entities)
    return (out_ent[:N_NODES], out_rel[:200])
